# baseline (speedup 1.0000x reference)
import functools
import jax
import jax.numpy as jnp
from jax import lax
from jax.experimental import pallas as pl
from jax.experimental.pallas import tpu as pltpu
from jax.experimental.pallas import tpu_sc as plsc

_MAX_EXP = 15
_NPOS = 16
_NVAL = 16
_CDIM = _NPOS * _NVAL


def _sc_build_m(boards_flat, B):
    info = plsc.get_sparse_core_info()
    nw = info.num_cores * info.num_subcores
    b_per_w = B // nw
    chunk = min(b_per_w, 256)
    n_chunks = b_per_w // chunk
    mesh = plsc.VectorSubcoreMesh(core_axis_name="c", subcore_axis_name="s")

    @functools.partial(
        pl.kernel,
        mesh=mesh,
        out_type=jax.ShapeDtypeStruct((B * _CDIM,), jnp.float32),
        scratch_types=[
            pltpu.VMEM((b_per_w * _NPOS,), jnp.int32),
            pltpu.VMEM((chunk * _CDIM,), jnp.float32),
        ],
        compiler_params=pltpu.CompilerParams(needs_layout_passes=False),
    )
    def sc_kernel(boards_hbm, m_hbm, boards_v, mbuf):
        wid = lax.axis_index("s") * info.num_cores + lax.axis_index("c")
        base = wid * b_per_w
        pltpu.sync_copy(boards_hbm.at[pl.ds(base * _NPOS, b_per_w * _NPOS)],
                        boards_v)
        colbase = lax.iota(jnp.int32, 16) * _NVAL
        zeros16 = jnp.zeros((16,), jnp.float32)
        ones16 = jnp.ones((16,), jnp.float32)

        def zero_body(j, _):
            for k in range(16):
                mbuf[pl.ds((j * 16 + k) * 16, 16)] = zeros16
            return 0
        lax.fori_loop(0, chunk * _NPOS // 16, zero_body, 0)

        def scatter_chunk(c, value):
            def body(i, _):
                e = boards_v[pl.ds((c * chunk + i) * _NPOS, _NPOS)]
                col = jnp.clip(e, 0, _MAX_EXP) + colbase + i * _CDIM
                plsc.store_scatter(mbuf, [col], value)
                return 0
            lax.fori_loop(0, chunk, body, 0)

        for c in range(n_chunks):
            scatter_chunk(c, ones16)
            pltpu.sync_copy(
                mbuf,
                m_hbm.at[pl.ds((base + c * chunk) * _CDIM, chunk * _CDIM)])
            if c + 1 < n_chunks:
                scatter_chunk(c, zeros16)

    return sc_kernel(boards_flat)


def _mlp_kernel(m_ref, table_ref, W1_ref, b1_ref, W2_ref, b2_ref,
                W3_ref, b3_ref, out_ref, T_ref):
    @pl.when(pl.program_id(0) == 0)
    def _build_T():
        tab = table_ref[:]
        for p in range(_NPOS):
            T_ref[pl.ds(p * _NVAL, _NVAL), :] = jnp.dot(
                tab, W1_ref[pl.ds(p * 32, 32), :],
                preferred_element_type=jnp.float32)

    h1 = jnp.maximum(
        jnp.dot(m_ref[:], T_ref[:], preferred_element_type=jnp.float32)
        + b1_ref[:], 0.0)
    h2 = jnp.maximum(
        jnp.dot(h1, W2_ref[:], preferred_element_type=jnp.float32)
        + b2_ref[:], 0.0)
    out_ref[:] = (jnp.dot(h2, W3_ref[:], preferred_element_type=jnp.float32)
                  + b3_ref[:])


def _tc_mlp(M, table, W1, b1, W2, b2, W3, b3, tile=1024):
    B = M.shape[0]
    return pl.pallas_call(
        _mlp_kernel,
        grid=(B // tile,),
        in_specs=[
            pl.BlockSpec((tile, _CDIM), lambda i: (i, 0)),
            pl.BlockSpec((_NVAL, 32), lambda i: (0, 0)),
            pl.BlockSpec((512, 256), lambda i: (0, 0)),
            pl.BlockSpec((1, 256), lambda i: (0, 0)),
            pl.BlockSpec((256, 256), lambda i: (0, 0)),
            pl.BlockSpec((1, 256), lambda i: (0, 0)),
            pl.BlockSpec((256, 4), lambda i: (0, 0)),
            pl.BlockSpec((1, 4), lambda i: (0, 0)),
        ],
        out_specs=pl.BlockSpec((tile, 4), lambda i: (i, 0)),
        out_shape=jax.ShapeDtypeStruct((B, 4), jnp.float32),
        scratch_shapes=[pltpu.VMEM((_CDIM, 256), jnp.float32)],
        compiler_params=pltpu.CompilerParams(
            dimension_semantics=("arbitrary",)),
    )(M, table, W1, b1.reshape(1, 256), W2, b2.reshape(1, 256), W3,
      b3.reshape(1, 4))


@jax.jit
def _run(boards, table, W1, b1, W2, b2, W3, b3):
    B = boards.shape[0]
    m_flat = _sc_build_m(boards.reshape(-1), B)
    M = m_flat.reshape(B, _CDIM)
    return _tc_mlp(M, table, W1, b1, W2, b2, W3, b3)


def kernel(boards, table, W1, b1, W2, b2, W3, b3):
    return _run(boards, table, W1, b1, W2, b2, W3, b3)


# speedup vs baseline: 1.3557x; 1.3557x over previous
import functools
import jax
import jax.numpy as jnp
from jax import lax
from jax.experimental import pallas as pl
from jax.experimental.pallas import tpu as pltpu
from jax.experimental.pallas import tpu_sc as plsc

_MAX_EXP = 15
_NPOS = 16
_NVAL = 16
_CDIM = _NPOS * _NVAL


def _sc_build_m(boards, B):
    info = plsc.get_sparse_core_info()
    nw = info.num_cores * info.num_subcores
    b_per_w = B // nw
    chunk = min(b_per_w, 256)
    n_chunks = b_per_w // chunk
    mesh = plsc.VectorSubcoreMesh(core_axis_name="c", subcore_axis_name="s")

    @functools.partial(
        pl.kernel,
        mesh=mesh,
        out_type=jax.ShapeDtypeStruct((B, _CDIM), jnp.float32),
        scratch_types=[
            pltpu.VMEM((b_per_w, _NPOS), jnp.int32),
            pltpu.VMEM((chunk, _CDIM), jnp.float32),
        ],
        compiler_params=pltpu.CompilerParams(needs_layout_passes=False),
    )
    def sc_kernel(boards_hbm, m_hbm, boards_v, mbuf):
        wid = lax.axis_index("s") * info.num_cores + lax.axis_index("c")
        base = wid * b_per_w
        pltpu.sync_copy(boards_hbm.at[pl.ds(base, b_per_w)], boards_v)
        colbase = lax.iota(jnp.int32, 16) * _NVAL
        zeros16 = jnp.zeros((16,), jnp.float32)
        ones16 = jnp.ones((16,), jnp.float32)

        def zero_body(r, _):
            for k in range(_CDIM // 16):
                mbuf[r, pl.ds(k * 16, 16)] = zeros16
            return 0
        lax.fori_loop(0, chunk, zero_body, 0)

        def scatter_chunk(c, value):
            def body(i, _):
                e = boards_v[c * chunk + i, :]
                col = jnp.clip(e, 0, _MAX_EXP) + colbase
                row = jnp.full((16,), i, jnp.int32)
                plsc.store_scatter(mbuf, [row, col], value)
                return 0
            lax.fori_loop(0, chunk, body, 0)

        for c in range(n_chunks):
            scatter_chunk(c, ones16)
            pltpu.sync_copy(mbuf, m_hbm.at[pl.ds(base + c * chunk, chunk)])
            if c + 1 < n_chunks:
                scatter_chunk(c, zeros16)

    return sc_kernel(boards)


def _mlp_kernel(m_ref, table_ref, W1_ref, b1_ref, W2_ref, b2_ref,
                W3_ref, b3_ref, out_ref, T_ref):
    @pl.when(pl.program_id(0) == 0)
    def _build_T():
        tab = table_ref[:]
        for p in range(_NPOS):
            T_ref[pl.ds(p * _NVAL, _NVAL), :] = jnp.dot(
                tab, W1_ref[pl.ds(p * 32, 32), :],
                preferred_element_type=jnp.float32)

    h1 = jnp.maximum(
        jnp.dot(m_ref[:], T_ref[:], preferred_element_type=jnp.float32)
        + b1_ref[:], 0.0)
    h2 = jnp.maximum(
        jnp.dot(h1, W2_ref[:], preferred_element_type=jnp.float32)
        + b2_ref[:], 0.0)
    out_ref[:] = (jnp.dot(h2, W3_ref[:], preferred_element_type=jnp.float32)
                  + b3_ref[:])


def _tc_mlp(M, table, W1, b1, W2, b2, W3, b3, tile=1024):
    B = M.shape[0]
    return pl.pallas_call(
        _mlp_kernel,
        grid=(B // tile,),
        in_specs=[
            pl.BlockSpec((tile, _CDIM), lambda i: (i, 0)),
            pl.BlockSpec((_NVAL, 32), lambda i: (0, 0)),
            pl.BlockSpec((512, 256), lambda i: (0, 0)),
            pl.BlockSpec((1, 256), lambda i: (0, 0)),
            pl.BlockSpec((256, 256), lambda i: (0, 0)),
            pl.BlockSpec((1, 256), lambda i: (0, 0)),
            pl.BlockSpec((256, 4), lambda i: (0, 0)),
            pl.BlockSpec((1, 4), lambda i: (0, 0)),
        ],
        out_specs=pl.BlockSpec((tile, 4), lambda i: (i, 0)),
        out_shape=jax.ShapeDtypeStruct((B, 4), jnp.float32),
        scratch_shapes=[pltpu.VMEM((_CDIM, 256), jnp.float32)],
        compiler_params=pltpu.CompilerParams(
            dimension_semantics=("arbitrary",)),
    )(M, table, W1, b1.reshape(1, 256), W2, b2.reshape(1, 256), W3,
      b3.reshape(1, 4))


@jax.jit
def _run(boards, table, W1, b1, W2, b2, W3, b3):
    B = boards.shape[0]
    M = _sc_build_m(boards, B)
    return _tc_mlp(M, table, W1, b1, W2, b2, W3, b3)


def kernel(boards, table, W1, b1, W2, b2, W3, b3):
    return _run(boards, table, W1, b1, W2, b2, W3, b3)


# speedup vs baseline: 1.3626x; 1.0051x over previous
import functools
import jax
import jax.numpy as jnp
from jax import lax
from jax.experimental import pallas as pl
from jax.experimental.pallas import tpu as pltpu
from jax.experimental.pallas import tpu_sc as plsc

_MAX_EXP = 15
_NPOS = 16
_NVAL = 16
_CDIM = _NPOS * _NVAL


def _sc_build_m(boards, B):
    info = plsc.get_sparse_core_info()
    nw = info.num_cores * info.num_subcores
    b_per_w = B // nw
    chunk = min(b_per_w, 256)
    n_chunks = b_per_w // chunk
    mesh = plsc.VectorSubcoreMesh(core_axis_name="c", subcore_axis_name="s")

    @functools.partial(
        pl.kernel,
        mesh=mesh,
        out_type=jax.ShapeDtypeStruct((B, _CDIM), jnp.float32),
        scratch_types=[
            pltpu.VMEM((b_per_w, _NPOS), jnp.int32),
            pltpu.VMEM((chunk, _CDIM), jnp.float32),
        ],
        compiler_params=pltpu.CompilerParams(needs_layout_passes=False,
                                             use_tc_tiling_on_sc=True),
    )
    def sc_kernel(boards_hbm, m_hbm, boards_v, mbuf):
        wid = lax.axis_index("s") * info.num_cores + lax.axis_index("c")
        base = wid * b_per_w
        pltpu.sync_copy(boards_hbm.at[pl.ds(base, b_per_w)], boards_v)
        colbase = lax.iota(jnp.int32, 16) * _NVAL
        zeros16 = jnp.zeros((16,), jnp.float32)
        ones16 = jnp.ones((16,), jnp.float32)

        def zero_body(r, _):
            for k in range(_CDIM // 16):
                mbuf[r, pl.ds(k * 16, 16)] = zeros16
            return 0
        lax.fori_loop(0, chunk, zero_body, 0)

        def scatter_chunk(c, value):
            def body(i, _):
                e = boards_v[c * chunk + i, :]
                col = jnp.clip(e, 0, _MAX_EXP) + colbase
                row = jnp.full((16,), i, jnp.int32)
                plsc.store_scatter(mbuf, [row, col], value)
                return 0
            lax.fori_loop(0, chunk, body, 0)

        for c in range(n_chunks):
            scatter_chunk(c, ones16)
            pltpu.sync_copy(mbuf, m_hbm.at[pl.ds(base + c * chunk, chunk)])
            if c + 1 < n_chunks:
                scatter_chunk(c, zeros16)

    return sc_kernel(boards)


def _mlp_kernel(m_ref, table_ref, W1_ref, b1_ref, W2_ref, b2_ref,
                W3_ref, b3_ref, out_ref, T_ref):
    @pl.when(pl.program_id(0) == 0)
    def _build_T():
        tab = table_ref[:]
        for p in range(_NPOS):
            T_ref[pl.ds(p * _NVAL, _NVAL), :] = jnp.dot(
                tab, W1_ref[pl.ds(p * 32, 32), :],
                preferred_element_type=jnp.float32)

    h1 = jnp.maximum(
        jnp.dot(m_ref[:], T_ref[:], preferred_element_type=jnp.float32)
        + b1_ref[:], 0.0)
    h2 = jnp.maximum(
        jnp.dot(h1, W2_ref[:], preferred_element_type=jnp.float32)
        + b2_ref[:], 0.0)
    out_ref[:] = (jnp.dot(h2, W3_ref[:], preferred_element_type=jnp.float32)
                  + b3_ref[:])


def _tc_mlp(M, table, W1, b1, W2, b2, W3, b3, tile=1024):
    B = M.shape[0]
    return pl.pallas_call(
        _mlp_kernel,
        grid=(B // tile,),
        in_specs=[
            pl.BlockSpec((tile, _CDIM), lambda i: (i, 0)),
            pl.BlockSpec((_NVAL, 32), lambda i: (0, 0)),
            pl.BlockSpec((512, 256), lambda i: (0, 0)),
            pl.BlockSpec((1, 256), lambda i: (0, 0)),
            pl.BlockSpec((256, 256), lambda i: (0, 0)),
            pl.BlockSpec((1, 256), lambda i: (0, 0)),
            pl.BlockSpec((256, 4), lambda i: (0, 0)),
            pl.BlockSpec((1, 4), lambda i: (0, 0)),
        ],
        out_specs=pl.BlockSpec((tile, 4), lambda i: (i, 0)),
        out_shape=jax.ShapeDtypeStruct((B, 4), jnp.float32),
        scratch_shapes=[pltpu.VMEM((_CDIM, 256), jnp.float32)],
        compiler_params=pltpu.CompilerParams(
            dimension_semantics=("arbitrary",)),
    )(M, table, W1, b1.reshape(1, 256), W2, b2.reshape(1, 256), W3,
      b3.reshape(1, 4))


@jax.jit
def _run(boards, table, W1, b1, W2, b2, W3, b3):
    B = boards.shape[0]
    M = _sc_build_m(boards, B)
    return _tc_mlp(M, table, W1, b1, W2, b2, W3, b3)


def kernel(boards, table, W1, b1, W2, b2, W3, b3):
    return _run(boards, table, W1, b1, W2, b2, W3, b3)


# speedup vs baseline: 1.4526x; 1.0661x over previous
import functools
import jax
import jax.numpy as jnp
from jax import lax
from jax.experimental import pallas as pl
from jax.experimental.pallas import tpu as pltpu
from jax.experimental.pallas import tpu_sc as plsc

_MAX_EXP = 15
_NPOS = 16
_NVAL = 16
_CDIM = _NPOS * _NVAL


def _sc_build_m(boards, B):
    info = plsc.get_sparse_core_info()
    nw = info.num_cores * info.num_subcores
    b_per_w = B // nw
    chunk = min(b_per_w, 256)
    n_chunks = b_per_w // chunk
    mesh = plsc.VectorSubcoreMesh(core_axis_name="c", subcore_axis_name="s")

    @functools.partial(
        pl.kernel,
        mesh=mesh,
        out_type=jax.ShapeDtypeStruct((B, _CDIM), jnp.float32),
        scratch_types=[
            pltpu.VMEM((b_per_w, _NPOS), jnp.int32),
            pltpu.VMEM((chunk, _CDIM), jnp.float32),
        ],
        compiler_params=pltpu.CompilerParams(needs_layout_passes=False,
                                             use_tc_tiling_on_sc=True),
    )
    def sc_kernel(boards_hbm, m_hbm, boards_v, mbuf):
        wid = lax.axis_index("s") * info.num_cores + lax.axis_index("c")
        base = wid * b_per_w
        pltpu.sync_copy(boards_hbm.at[pl.ds(base, b_per_w)], boards_v)
        colbase = lax.iota(jnp.int32, 16) * _NVAL
        zeros16 = jnp.zeros((16,), jnp.float32)
        ones16 = jnp.ones((16,), jnp.float32)

        def zero_body(r, _):
            for k in range(_CDIM // 16):
                mbuf[r, pl.ds(k * 16, 16)] = zeros16
            return 0
        lax.fori_loop(0, chunk, zero_body, 0)

        def scatter_chunk(c, value):
            def body(i, _):
                e = boards_v[c * chunk + i, :]
                col = jnp.clip(e, 0, _MAX_EXP) + colbase
                row = jnp.full((16,), i, jnp.int32)
                plsc.store_scatter(mbuf, [row, col], value)
                return 0
            lax.fori_loop(0, chunk, body, 0)

        for c in range(n_chunks):
            scatter_chunk(c, ones16)
            pltpu.sync_copy(mbuf, m_hbm.at[pl.ds(base + c * chunk, chunk)])
            if c + 1 < n_chunks:
                scatter_chunk(c, zeros16)

    return sc_kernel(boards)


def _mlp_kernel(m_ref, table_ref, W1_ref, b1_ref, W2_ref, b2_ref,
                W3_ref, b3_ref, out_ref, T_ref):
    @pl.when(pl.program_id(0) == 0)
    def _build_T():
        tab = table_ref[:]
        for p in range(_NPOS):
            T_ref[pl.ds(p * _NVAL, _NVAL), :] = jnp.dot(
                tab, W1_ref[pl.ds(p * 32, 32), :],
                preferred_element_type=jnp.float32)

    h1 = jnp.maximum(
        jnp.dot(m_ref[:], T_ref[:], preferred_element_type=jnp.float32)
        + b1_ref[:], 0.0)
    h2 = jnp.maximum(
        jnp.dot(h1, W2_ref[:], preferred_element_type=jnp.float32)
        + b2_ref[:], 0.0)
    out_ref[:] = (jnp.dot(h2, W3_ref[:], preferred_element_type=jnp.float32)
                  + b3_ref[:])


def _tc_mlp(M, table, W1, b1, W2, b2, W3, b3, tile=1024):
    B = M.shape[0]
    return pl.pallas_call(
        _mlp_kernel,
        grid=(B // tile,),
        in_specs=[
            pl.BlockSpec((tile, _CDIM), lambda i: (i, 0)),
            pl.BlockSpec((_NVAL, 32), lambda i: (0, 0)),
            pl.BlockSpec((512, 256), lambda i: (0, 0)),
            pl.BlockSpec((1, 256), lambda i: (0, 0)),
            pl.BlockSpec((256, 256), lambda i: (0, 0)),
            pl.BlockSpec((1, 256), lambda i: (0, 0)),
            pl.BlockSpec((256, 4), lambda i: (0, 0)),
            pl.BlockSpec((1, 4), lambda i: (0, 0)),
        ],
        out_specs=pl.BlockSpec((tile, 4), lambda i: (i, 0)),
        out_shape=jax.ShapeDtypeStruct((B, 4), jnp.float32),
        scratch_shapes=[pltpu.VMEM((_CDIM, 256), jnp.float32)],
        compiler_params=pltpu.CompilerParams(
            dimension_semantics=("arbitrary",)),
    )(M, table, W1, b1.reshape(1, 256), W2, b2.reshape(1, 256), W3,
      b3.reshape(1, 4))


@jax.jit
def _run(boards, table, W1, b1, W2, b2, W3, b3):
    B = boards.shape[0]
    n_overlap = 2
    Bc = B // n_overlap
    Ms = [_sc_build_m(lax.slice_in_dim(boards, c * Bc, (c + 1) * Bc), Bc)
          for c in range(n_overlap)]
    outs = [_tc_mlp(m, table, W1, b1, W2, b2, W3, b3) for m in Ms]
    return jnp.concatenate(outs, axis=0)


def kernel(boards, table, W1, b1, W2, b2, W3, b3):
    return _run(boards, table, W1, b1, W2, b2, W3, b3)


# speedup vs baseline: 1.7377x; 1.1963x over previous
import functools
import jax
import jax.numpy as jnp
from jax import lax
from jax.experimental import pallas as pl
from jax.experimental.pallas import tpu as pltpu
from jax.experimental.pallas import tpu_sc as plsc

_MAX_EXP = 15
_NPOS = 16
_NVAL = 16
_CDIM = _NPOS * _NVAL


def _sc_build_m(boards, B):
    info = plsc.get_sparse_core_info()
    nw = info.num_cores * info.num_subcores
    b_per_w = B // nw
    chunk = min(b_per_w, 256)
    n_chunks = b_per_w // chunk
    mesh = plsc.VectorSubcoreMesh(core_axis_name="c", subcore_axis_name="s")

    @functools.partial(
        pl.kernel,
        mesh=mesh,
        out_type=jax.ShapeDtypeStruct((B, _CDIM), jnp.float32),
        scratch_types=[
            pltpu.VMEM((b_per_w, _NPOS), jnp.int32),
            pltpu.VMEM((chunk, _CDIM), jnp.float32),
        ],
        compiler_params=pltpu.CompilerParams(needs_layout_passes=False,
                                             use_tc_tiling_on_sc=True),
    )
    def sc_kernel(boards_hbm, m_hbm, boards_v, mbuf):
        wid = lax.axis_index("s") * info.num_cores + lax.axis_index("c")
        base = wid * b_per_w
        pltpu.sync_copy(boards_hbm.at[pl.ds(base, b_per_w)], boards_v)
        colbase = lax.iota(jnp.int32, 16) * _NVAL
        zeros16 = jnp.zeros((16,), jnp.float32)
        ones16 = jnp.ones((16,), jnp.float32)

        def zero_body(r, _):
            for k in range(_CDIM // 16):
                mbuf[r, pl.ds(k * 16, 16)] = zeros16
            return 0
        lax.fori_loop(0, chunk, zero_body, 0)

        def scatter_chunk(c, value):
            def body(i, _):
                e = boards_v[c * chunk + i, :]
                col = jnp.clip(e, 0, _MAX_EXP) + colbase
                row = jnp.full((16,), i, jnp.int32)
                plsc.store_scatter(mbuf, [row, col], value)
                return 0
            lax.fori_loop(0, chunk, body, 0)

        for c in range(n_chunks):
            scatter_chunk(c, ones16)
            pltpu.sync_copy(mbuf, m_hbm.at[pl.ds(base + c * chunk, chunk)])
            if c + 1 < n_chunks:
                scatter_chunk(c, zeros16)

    return sc_kernel(boards)


def _lookup_mlp_kernel(boards_ref, table_ref, W1_ref, b1_ref, W2_ref, b2_ref,
                       W3_ref, b3_ref, out_ref, T_ref):
    tile = boards_ref.shape[0]

    @pl.when(pl.program_id(0) == 0)
    def _build_T():
        tab = table_ref[:]
        for p in range(_NPOS):
            T_ref[pl.ds(p * _NVAL, _NVAL), :] = jnp.dot(
                tab, W1_ref[pl.ds(p * 32, 32), :],
                preferred_element_type=jnp.float32)

    enc = jnp.clip(boards_ref[:], 0, _MAX_EXP)

    colid = lax.broadcasted_iota(jnp.int32, (tile, _CDIM), 1)
    sel = (lax.broadcasted_iota(jnp.int32, (_NPOS, _CDIM), 0)
           == lax.broadcasted_iota(jnp.int32, (_NPOS, _CDIM), 1) // _NVAL)
    rep = jnp.dot(enc.astype(jnp.float32), sel.astype(jnp.float32),
                  preferred_element_type=jnp.float32)
    M = jnp.where(rep == (colid % _NVAL).astype(jnp.float32), 1.0, 0.0)

    h1 = jnp.maximum(
        jnp.dot(M, T_ref[:], preferred_element_type=jnp.float32) + b1_ref[:],
        0.0)
    h2 = jnp.maximum(
        jnp.dot(h1, W2_ref[:], preferred_element_type=jnp.float32) + b2_ref[:],
        0.0)
    out_ref[:] = (jnp.dot(h2, W3_ref[:], preferred_element_type=jnp.float32)
                  + b3_ref[:])


def _tc_full(boards, table, W1, b1, W2, b2, W3, b3, tile=1024):
    B = boards.shape[0]
    return pl.pallas_call(
        _lookup_mlp_kernel,
        grid=(B // tile,),
        in_specs=[
            pl.BlockSpec((tile, _NPOS), lambda i: (i, 0)),
            pl.BlockSpec((_NVAL, 32), lambda i: (0, 0)),
            pl.BlockSpec((512, 256), lambda i: (0, 0)),
            pl.BlockSpec((1, 256), lambda i: (0, 0)),
            pl.BlockSpec((256, 256), lambda i: (0, 0)),
            pl.BlockSpec((1, 256), lambda i: (0, 0)),
            pl.BlockSpec((256, 4), lambda i: (0, 0)),
            pl.BlockSpec((1, 4), lambda i: (0, 0)),
        ],
        out_specs=pl.BlockSpec((tile, 4), lambda i: (i, 0)),
        out_shape=jax.ShapeDtypeStruct((B, 4), jnp.float32),
        scratch_shapes=[pltpu.VMEM((_CDIM, 256), jnp.float32)],
        compiler_params=pltpu.CompilerParams(
            dimension_semantics=("arbitrary",)),
    )(boards, table, W1, b1.reshape(1, 256), W2, b2.reshape(1, 256), W3,
      b3.reshape(1, 4))


def _mlp_kernel(m_ref, table_ref, W1_ref, b1_ref, W2_ref, b2_ref,
                W3_ref, b3_ref, out_ref, T_ref):
    @pl.when(pl.program_id(0) == 0)
    def _build_T():
        tab = table_ref[:]
        for p in range(_NPOS):
            T_ref[pl.ds(p * _NVAL, _NVAL), :] = jnp.dot(
                tab, W1_ref[pl.ds(p * 32, 32), :],
                preferred_element_type=jnp.float32)

    h1 = jnp.maximum(
        jnp.dot(m_ref[:], T_ref[:], preferred_element_type=jnp.float32)
        + b1_ref[:], 0.0)
    h2 = jnp.maximum(
        jnp.dot(h1, W2_ref[:], preferred_element_type=jnp.float32)
        + b2_ref[:], 0.0)
    out_ref[:] = (jnp.dot(h2, W3_ref[:], preferred_element_type=jnp.float32)
                  + b3_ref[:])


def _tc_mlp(M, table, W1, b1, W2, b2, W3, b3, tile=1024):
    B = M.shape[0]
    return pl.pallas_call(
        _mlp_kernel,
        grid=(B // tile,),
        in_specs=[
            pl.BlockSpec((tile, _CDIM), lambda i: (i, 0)),
            pl.BlockSpec((_NVAL, 32), lambda i: (0, 0)),
            pl.BlockSpec((512, 256), lambda i: (0, 0)),
            pl.BlockSpec((1, 256), lambda i: (0, 0)),
            pl.BlockSpec((256, 256), lambda i: (0, 0)),
            pl.BlockSpec((1, 256), lambda i: (0, 0)),
            pl.BlockSpec((256, 4), lambda i: (0, 0)),
            pl.BlockSpec((1, 4), lambda i: (0, 0)),
        ],
        out_specs=pl.BlockSpec((tile, 4), lambda i: (i, 0)),
        out_shape=jax.ShapeDtypeStruct((B, 4), jnp.float32),
        scratch_shapes=[pltpu.VMEM((_CDIM, 256), jnp.float32)],
        compiler_params=pltpu.CompilerParams(
            dimension_semantics=("arbitrary",)),
    )(M, table, W1, b1.reshape(1, 256), W2, b2.reshape(1, 256), W3,
      b3.reshape(1, 4))


@jax.jit
def _run(boards, table, W1, b1, W2, b2, W3, b3):
    B = boards.shape[0]
    B_sc = (3 * B // 8 // 2048) * 2048
    B_tc = B - B_sc
    M_sc = _sc_build_m(lax.slice_in_dim(boards, B_tc, B), B_sc)
    out_tc = _tc_full(lax.slice_in_dim(boards, 0, B_tc), table, W1, b1, W2,
                      b2, W3, b3)
    out_sc = _tc_mlp(M_sc, table, W1, b1, W2, b2, W3, b3)
    return jnp.concatenate([out_tc, out_sc], axis=0)


def kernel(boards, table, W1, b1, W2, b2, W3, b3):
    return _run(boards, table, W1, b1, W2, b2, W3, b3)


# speedup vs baseline: 1.9192x; 1.1044x over previous
import functools
import jax
import jax.numpy as jnp
from jax import lax
from jax.experimental import pallas as pl
from jax.experimental.pallas import tpu as pltpu
from jax.experimental.pallas import tpu_sc as plsc

_MAX_EXP = 15
_NPOS = 16
_NVAL = 16
_CDIM = _NPOS * _NVAL


def _sc_build_m(boards, B):
    info = plsc.get_sparse_core_info()
    nw = info.num_cores * info.num_subcores
    b_per_w = B // nw
    chunk = min(b_per_w, 256)
    n_chunks = b_per_w // chunk
    mesh = plsc.VectorSubcoreMesh(core_axis_name="c", subcore_axis_name="s")

    @functools.partial(
        pl.kernel,
        mesh=mesh,
        out_type=jax.ShapeDtypeStruct((B, _CDIM), jnp.float32),
        scratch_types=[
            pltpu.VMEM((b_per_w, _NPOS), jnp.int32),
            pltpu.VMEM((chunk, _CDIM), jnp.float32),
        ],
        compiler_params=pltpu.CompilerParams(needs_layout_passes=False,
                                             use_tc_tiling_on_sc=True),
    )
    def sc_kernel(boards_hbm, m_hbm, boards_v, mbuf):
        wid = lax.axis_index("s") * info.num_cores + lax.axis_index("c")
        base = wid * b_per_w
        pltpu.sync_copy(boards_hbm.at[pl.ds(base, b_per_w)], boards_v)
        colbase = lax.iota(jnp.int32, 16) * _NVAL
        zeros16 = jnp.zeros((16,), jnp.float32)
        ones16 = jnp.ones((16,), jnp.float32)

        def zero_body(r, _):
            for k in range(_CDIM // 16):
                mbuf[r, pl.ds(k * 16, 16)] = zeros16
            return 0
        lax.fori_loop(0, chunk, zero_body, 0)

        def scatter_chunk(c, value):
            def body(i, _):
                e = boards_v[c * chunk + i, :]
                col = jnp.clip(e, 0, _MAX_EXP) + colbase
                row = jnp.full((16,), i, jnp.int32)
                plsc.store_scatter(mbuf, [row, col], value)
                return 0
            lax.fori_loop(0, chunk, body, 0)

        for c in range(n_chunks):
            scatter_chunk(c, ones16)
            pltpu.sync_copy(mbuf, m_hbm.at[pl.ds(base + c * chunk, chunk)])
            if c + 1 < n_chunks:
                scatter_chunk(c, zeros16)

    return sc_kernel(boards)


def _lookup_mlp_kernel(boards_ref, table_ref, W1_ref, b1_ref, W2_ref, b2_ref,
                       W3_ref, b3_ref, out_ref, T_ref):
    tile = boards_ref.shape[0]

    @pl.when(pl.program_id(0) == 0)
    def _build_T():
        tab = table_ref[:]
        for p in range(_NPOS):
            T_ref[pl.ds(p * _NVAL, _NVAL), :] = jnp.dot(
                tab, W1_ref[pl.ds(p * 32, 32), :],
                preferred_element_type=jnp.float32)

    enc = jnp.clip(boards_ref[:], 0, _MAX_EXP)

    colid = lax.broadcasted_iota(jnp.int32, (tile, _CDIM), 1)
    sel = (lax.broadcasted_iota(jnp.int32, (_NPOS, _CDIM), 0)
           == lax.broadcasted_iota(jnp.int32, (_NPOS, _CDIM), 1) // _NVAL)
    rep = jnp.dot(enc.astype(jnp.float32), sel.astype(jnp.float32),
                  preferred_element_type=jnp.float32)
    M = jnp.where(rep == (colid % _NVAL).astype(jnp.float32), 1.0, 0.0)

    h1 = jnp.maximum(
        jnp.dot(M, T_ref[:], preferred_element_type=jnp.float32) + b1_ref[:],
        0.0)
    h2 = jnp.maximum(
        jnp.dot(h1, W2_ref[:], preferred_element_type=jnp.float32) + b2_ref[:],
        0.0)
    out_ref[:] = (jnp.dot(h2, W3_ref[:], preferred_element_type=jnp.float32)
                  + b3_ref[:])


def _tc_full(boards, table, W1, b1, W2, b2, W3, b3, B_out, tile=2048):
    B = boards.shape[0]
    return pl.pallas_call(
        _lookup_mlp_kernel,
        grid=(B // tile,),
        in_specs=[
            pl.BlockSpec((tile, _NPOS), lambda i: (i, 0)),
            pl.BlockSpec((_NVAL, 32), lambda i: (0, 0)),
            pl.BlockSpec((512, 256), lambda i: (0, 0)),
            pl.BlockSpec((1, 256), lambda i: (0, 0)),
            pl.BlockSpec((256, 256), lambda i: (0, 0)),
            pl.BlockSpec((1, 256), lambda i: (0, 0)),
            pl.BlockSpec((256, 4), lambda i: (0, 0)),
            pl.BlockSpec((1, 4), lambda i: (0, 0)),
        ],
        out_specs=pl.BlockSpec((tile, 4), lambda i: (i, 0)),
        out_shape=jax.ShapeDtypeStruct((B_out, 4), jnp.float32),
        scratch_shapes=[pltpu.VMEM((_CDIM, 256), jnp.float32)],
        compiler_params=pltpu.CompilerParams(
            dimension_semantics=("arbitrary",)),
    )(boards, table, W1, b1.reshape(1, 256), W2, b2.reshape(1, 256), W3,
      b3.reshape(1, 4))


def _mlp_kernel(m_ref, table_ref, W1_ref, b1_ref, W2_ref, b2_ref,
                W3_ref, b3_ref, out_ref, T_ref):
    @pl.when(pl.program_id(0) == 0)
    def _build_T():
        tab = table_ref[:]
        for p in range(_NPOS):
            T_ref[pl.ds(p * _NVAL, _NVAL), :] = jnp.dot(
                tab, W1_ref[pl.ds(p * 32, 32), :],
                preferred_element_type=jnp.float32)

    h1 = jnp.maximum(
        jnp.dot(m_ref[:], T_ref[:], preferred_element_type=jnp.float32)
        + b1_ref[:], 0.0)
    h2 = jnp.maximum(
        jnp.dot(h1, W2_ref[:], preferred_element_type=jnp.float32)
        + b2_ref[:], 0.0)
    out_ref[:] = (jnp.dot(h2, W3_ref[:], preferred_element_type=jnp.float32)
                  + b3_ref[:])


def _mlp_kernel_alias(m_ref, table_ref, W1_ref, b1_ref, W2_ref, b2_ref,
                      W3_ref, b3_ref, prev_ref, out_ref, T_ref):
    del prev_ref
    _mlp_kernel(m_ref, table_ref, W1_ref, b1_ref, W2_ref, b2_ref,
                W3_ref, b3_ref, out_ref, T_ref)


def _tc_mlp(M, table, W1, b1, W2, b2, W3, b3, prev_out, row0, tile=2048):
    B = M.shape[0]
    off = row0 // tile
    return pl.pallas_call(
        _mlp_kernel_alias,
        grid=(B // tile,),
        in_specs=[
            pl.BlockSpec((tile, _CDIM), lambda i: (i, 0)),
            pl.BlockSpec((_NVAL, 32), lambda i: (0, 0)),
            pl.BlockSpec((512, 256), lambda i: (0, 0)),
            pl.BlockSpec((1, 256), lambda i: (0, 0)),
            pl.BlockSpec((256, 256), lambda i: (0, 0)),
            pl.BlockSpec((1, 256), lambda i: (0, 0)),
            pl.BlockSpec((256, 4), lambda i: (0, 0)),
            pl.BlockSpec((1, 4), lambda i: (0, 0)),
            pl.BlockSpec(memory_space=pltpu.MemorySpace.HBM),
        ],
        out_specs=pl.BlockSpec((tile, 4), lambda i: (i + off, 0)),
        out_shape=jax.ShapeDtypeStruct(prev_out.shape, jnp.float32),
        scratch_shapes=[pltpu.VMEM((_CDIM, 256), jnp.float32)],
        input_output_aliases={8: 0},
        compiler_params=pltpu.CompilerParams(
            dimension_semantics=("arbitrary",)),
    )(M, table, W1, b1.reshape(1, 256), W2, b2.reshape(1, 256), W3,
      b3.reshape(1, 4), prev_out)


@jax.jit
def _run(boards, table, W1, b1, W2, b2, W3, b3):
    B = boards.shape[0]
    B_sc = (3 * B // 8 // 2048) * 2048
    B_tc = B - B_sc
    M_sc = _sc_build_m(lax.slice_in_dim(boards, B_tc, B), B_sc)
    out_tc = _tc_full(lax.slice_in_dim(boards, 0, B_tc), table, W1, b1, W2,
                      b2, W3, b3, B_out=B)
    return _tc_mlp(M_sc, table, W1, b1, W2, b2, W3, b3, out_tc, B_tc)


def kernel(boards, table, W1, b1, W2, b2, W3, b3):
    return _run(boards, table, W1, b1, W2, b2, W3, b3)
